# baseline (speedup 1.0000x reference)
import functools

import jax
import jax.numpy as jnp
from jax import lax
from jax.experimental import pallas as pl
from jax.experimental.pallas import tpu as pltpu
from jax.experimental.pallas import tpu_sc as plsc

D = 64
DP = 128
K = 5
NC = 2
NS = 16
NW = NC * NS
LANES = 16


def _sc_dots(v_i, v_j, neg_t, nodes2, ctx2):
    B = v_i.shape[0]
    PB = B // NW
    C = min(128, PB)
    NCHUNK = PB // C

    mesh = plsc.VectorSubcoreMesh(core_axis_name="c", subcore_axis_name="s")

    @functools.partial(
        pl.kernel,
        mesh=mesh,
        compiler_params=pltpu.CompilerParams(needs_layout_passes=False),
        out_type=jax.ShapeDtypeStruct((1 + K, B), jnp.float32),
        scratch_types=[
            pltpu.VMEM((C,), jnp.int32),
            pltpu.VMEM((C,), jnp.int32),
            pltpu.VMEM((K * C,), jnp.int32),
            pltpu.VMEM((C,), jnp.int32),
            pltpu.VMEM((C,), jnp.int32),
            pltpu.VMEM((K * C,), jnp.int32),
            pltpu.VMEM((C, DP), jnp.float32),
            pltpu.VMEM((C, DP), jnp.float32),
            pltpu.VMEM((K * C, DP), jnp.float32),
            pltpu.VMEM((1 + K, C), jnp.float32),
            pltpu.SemaphoreType.DMA,
        ],
    )
    def body(vi_hbm, vj_hbm, negt_hbm, nodes_hbm, ctx_hbm, out_hbm,
             vi_idx, vj_idx, neg_idx, vi_q, vj_q, neg_q,
             vi_rows, vj_rows, neg_rows, dots, sem):
        wid = lax.axis_index("s") * NC + lax.axis_index("c")
        lane = lax.iota(jnp.int32, 16)

        def split_q(idx_ref, q_ref, n):
            def gb(g, carry):
                q_ref[pl.ds(g * 16, 16)] = idx_ref[pl.ds(g * 16, 16)] >> 1
                return carry
            lax.fori_loop(0, n // 16, gb, 0)

        def chunk_body(ci, carry):
            base = wid * PB + ci * C
            pltpu.sync_copy(vi_hbm.at[pl.ds(base, C)], vi_idx)
            pltpu.sync_copy(vj_hbm.at[pl.ds(base, C)], vj_idx)
            for k in range(K):
                pltpu.sync_copy(negt_hbm.at[pl.ds(k * B + base, C)],
                                neg_idx.at[pl.ds(k * C, C)])
            split_q(vi_idx, vi_q, C)
            split_q(vj_idx, vj_q, C)
            split_q(neg_idx, neg_q, K * C)
            copies = [
                pltpu.async_copy(nodes_hbm.at[vi_q], vi_rows, sem),
                pltpu.async_copy(ctx_hbm.at[vj_q], vj_rows, sem),
            ]
            for k in range(K):
                copies.append(
                    pltpu.async_copy(ctx_hbm.at[neg_q.at[pl.ds(k * C, C)]],
                                     neg_rows.at[pl.ds(k * C, C)], sem))
            for c in copies:
                c.wait()

            def group_body(g, carry2):
                elem = g * 16 + lane
                off_i = (vi_idx[pl.ds(g * 16, 16)] & 1) * D
                off_j = (vj_idx[pl.ds(g * 16, 16)] & 1) * D
                off_n = [(neg_idx[pl.ds(k * C + g * 16, 16)] & 1) * D
                         for k in range(K)]
                nelem = [k * C + g * 16 + lane for k in range(K)]
                pos = jnp.zeros((16,), jnp.float32)
                neg = [jnp.zeros((16,), jnp.float32) for _ in range(K)]
                for w in range(D):
                    vv = plsc.load_gather(vi_rows, [elem, off_i + w])
                    jv = plsc.load_gather(vj_rows, [elem, off_j + w])
                    pos = pos + vv * jv
                    for k in range(K):
                        nv = plsc.load_gather(neg_rows, [nelem[k], off_n[k] + w])
                        neg[k] = neg[k] - vv * nv
                dots[0, pl.ds(g * 16, 16)] = pos
                for k in range(K):
                    dots[1 + k, pl.ds(g * 16, 16)] = neg[k]
                return carry2

            lax.fori_loop(0, C // 16, group_body, 0)
            pltpu.sync_copy(
                dots, out_hbm.at[:, pl.ds(pl.multiple_of(base, 128), C)])
            return carry

        lax.fori_loop(0, NCHUNK, chunk_body, 0)

    return body(v_i, v_j, neg_t, nodes2, ctx2)


def _tc_loss(dots2d, batch):
    def body(x_ref, o_ref):
        x = x_ref[...]
        ls = jnp.minimum(x, 0.0) - jnp.log1p(jnp.exp(-jnp.abs(x)))
        o_ref[0, 0] = -jnp.sum(ls) / batch

    return pl.pallas_call(
        body,
        out_shape=jax.ShapeDtypeStruct((1, 1), jnp.float32),
        out_specs=pl.BlockSpec(memory_space=pltpu.SMEM),
    )(dots2d)


def kernel(v_i, v_j, negsamples, device, nodes_embeddings, contextnodes_embeddings):
    B = v_i.shape[0]
    vi = v_i.astype(jnp.int32)
    vj = v_j.astype(jnp.int32)
    neg_t = negsamples.astype(jnp.int32).T.reshape(-1)
    n2 = nodes_embeddings.reshape(-1, DP)
    c2 = contextnodes_embeddings.reshape(-1, DP)
    dots = _sc_dots(vi, vj, neg_t, n2, c2)
    loss = _tc_loss(dots.reshape((1 + K) * B // 128, 128), B)
    return loss[0, 0]
